# baseline (speedup 1.0000x reference)
import jax
import jax.numpy as jnp
from jax.experimental import pallas as pl

_ALPHA = 0.1


def _soft_thres_body(x_ref, o_ref):
    x = x_ref[...]
    o_ref[...] = jnp.where(x > -_ALPHA, x - _ALPHA, x + _ALPHA)


def kernel(X):
    M, N = X.shape
    BM = 256
    return pl.pallas_call(
        _soft_thres_body,
        grid=(M // BM,),
        in_specs=[pl.BlockSpec((BM, N), lambda i: (i, 0))],
        out_specs=pl.BlockSpec((BM, N), lambda i: (i, 0)),
        out_shape=jax.ShapeDtypeStruct((M, N), X.dtype),
    )(X)


# speedup vs baseline: 1.0148x; 1.0148x over previous
import jax
import jax.numpy as jnp
from jax.experimental import pallas as pl

_ALPHA = 0.1


def _soft_thres_body(x_ref, o_ref):
    x = x_ref[...]
    o_ref[...] = jnp.where(x > -_ALPHA, x - _ALPHA, x + _ALPHA)


def kernel(X):
    M, N = X.shape
    BM = 512
    return pl.pallas_call(
        _soft_thres_body,
        grid=(M // BM,),
        in_specs=[pl.BlockSpec((BM, N), lambda i: (i, 0))],
        out_specs=pl.BlockSpec((BM, N), lambda i: (i, 0)),
        out_shape=jax.ShapeDtypeStruct((M, N), X.dtype),
    )(X)
